# baseline (speedup 1.0000x reference)
import functools

import jax
import jax.numpy as jnp
from jax import lax
from jax.experimental import pallas as pl
from jax.experimental.pallas import tpu as pltpu
from jax.experimental.pallas import tpu_sc as plsc

N_TOKEN = 1000000
D_EMBED = 64
EMB_SCALE = D_EMBED ** 0.5

_S = 16384
_T = 50
_N = _S * _T
_NW = 32
_TPW = _N // _NW
_C = 128
_NCH = _TPW // _C
_LANES = 16
_PAIR_W = 2 * D_EMBED


def _emb_body(idx_hbm, table_hbm, out_hbm,
              idx_all, pair0, pair1, off0, off1, rows0, rows1, out_s,
              sem0, sem1):
    wid = lax.axis_index("s") * 2 + lax.axis_index("c")
    t0 = pl.multiple_of(wid * _TPW, _TPW)

    pltpu.sync_copy(idx_hbm.at[pl.ds(t0, _TPW)], idx_all)

    def stage_fire(ch, pair_v, off_v, rows_v, sem):
        def prep(k, _):
            v = idx_all[pl.ds(ch * _C + k * _LANES, _LANES)]
            pair_v[pl.ds(k * _LANES, _LANES)] = v >> 1
            off_v[pl.ds(k * _LANES, _LANES)] = (v & 1) * D_EMBED
            return 0

        lax.fori_loop(0, _C // _LANES, prep, 0)
        pltpu.async_copy(table_hbm.at[pair_v], rows_v, sem)

    def process(ch, pair_v, off_v, rows_v, sem):
        pltpu.make_async_copy(table_hbm.at[pair_v], rows_v, sem).wait()

        def extract(g, _):
            ov = off_v[pl.ds(g * _LANES, _LANES)]
            for i in range(_LANES):
                t = g * _LANES + i
                o = ov[i]
                for k in range(D_EMBED // _LANES):
                    out_s[t, pl.ds(k * _LANES, _LANES)] = (
                        rows_v[t, pl.ds(o + k * _LANES, _LANES)] * EMB_SCALE
                    )
            return 0

        lax.fori_loop(0, _C // _LANES, extract, 0)
        pltpu.sync_copy(out_s, out_hbm.at[pl.ds(t0 + ch * _C, _C)])

    stage_fire(0, pair0, off0, rows0, sem0)
    stage_fire(1, pair1, off1, rows1, sem1)

    def pair_body(j, _):
        c0 = j * 2
        process(c0, pair0, off0, rows0, sem0)

        @pl.when(c0 + 2 < _NCH)
        def _():
            stage_fire(c0 + 2, pair0, off0, rows0, sem0)

        process(c0 + 1, pair1, off1, rows1, sem1)

        @pl.when(c0 + 3 < _NCH)
        def _():
            stage_fire(c0 + 3, pair1, off1, rows1, sem1)

        return 0

    lax.fori_loop(0, _NCH // 2, pair_body, 0)


_mesh = plsc.VectorSubcoreMesh(core_axis_name="c", subcore_axis_name="s")

_emb_call = functools.partial(
    pl.kernel,
    mesh=_mesh,
    out_type=jax.ShapeDtypeStruct((_N, D_EMBED), jnp.float32),
    scratch_types=[
        pltpu.VMEM((_TPW,), jnp.int32),
        pltpu.VMEM((_C,), jnp.int32),
        pltpu.VMEM((_C,), jnp.int32),
        pltpu.VMEM((_C,), jnp.int32),
        pltpu.VMEM((_C,), jnp.int32),
        pltpu.VMEM((_C, _PAIR_W), jnp.float32),
        pltpu.VMEM((_C, _PAIR_W), jnp.float32),
        pltpu.VMEM((_C, D_EMBED), jnp.float32),
        pltpu.SemaphoreType.DMA,
        pltpu.SemaphoreType.DMA,
    ],
)(_emb_body)


@jax.jit
def kernel(inp, emb_weight):
    idx = inp.reshape(-1).astype(jnp.int32)
    table_pairs = emb_weight.reshape(N_TOKEN // 2, _PAIR_W)
    out = _emb_call(idx, table_pairs)
    return out.reshape(_S, _T, D_EMBED)


# speedup vs baseline: 1.6323x; 1.6323x over previous
import functools

import jax
import jax.numpy as jnp
from jax import lax
from jax.experimental import pallas as pl
from jax.experimental.pallas import tpu as pltpu
from jax.experimental.pallas import tpu_sc as plsc

N_TOKEN = 1000000
D_EMBED = 64
EMB_SCALE = D_EMBED ** 0.5

_S = 16384
_T = 50
_NW = 32
_S_PER_W = _S // _NW
_C = 16
_TOK = _C * _T
_NCH = _S_PER_W // _C
_G = 16
_NG = _TOK // _G
_WG = 16
_LANES = 16


def _emb_body(idx_hbm, table_hbm, out_hbm, idx_v, rows_v, sem):
    wid = lax.axis_index("s") * 2 + lax.axis_index("c")
    s0 = wid * _S_PER_W

    def chunk_body(g, _):
        sb = s0 + g * _C
        tb = sb * _T
        pltpu.sync_copy(idx_hbm.at[pl.ds(pl.multiple_of(tb, _TOK), _TOK)], idx_v)

        def fire_group(q, _):
            v = idx_v[pl.ds(q * _G, _G)]
            for k in range(_G):
                pltpu.async_copy(
                    table_hbm.at[pl.ds(v[k], 1)],
                    rows_v.at[pl.ds(q * _G + k, 1)],
                    sem,
                )

            @pl.when(q >= _WG)
            def _():
                pltpu.make_async_copy(
                    table_hbm.at[pl.ds(0, _G)],
                    rows_v.at[pl.ds(0, _G)],
                    sem,
                ).wait()

            return 0

        lax.fori_loop(0, _NG, fire_group, 0)

        pltpu.make_async_copy(
            table_hbm.at[pl.ds(0, _WG * _G)],
            rows_v.at[pl.ds(0, _WG * _G)],
            sem,
        ).wait()

        def scale_row(r, _):
            for k in range(D_EMBED // _LANES):
                sl = pl.ds(k * _LANES, _LANES)
                rows_v[r, sl] = rows_v[r, sl] * EMB_SCALE
            return 0

        lax.fori_loop(0, _TOK, scale_row, 0)

        def write_sample(c, _):
            pltpu.sync_copy(
                rows_v.at[pl.ds(c * _T, _T)],
                out_hbm.at[sb + c],
            )
            return 0

        lax.fori_loop(0, _C, write_sample, 0)
        return 0

    lax.fori_loop(0, _NCH, chunk_body, 0)


_mesh = plsc.VectorSubcoreMesh(core_axis_name="c", subcore_axis_name="s")

_emb_call = functools.partial(
    pl.kernel,
    mesh=_mesh,
    out_type=jax.ShapeDtypeStruct((_S, _T, D_EMBED), jnp.float32),
    scratch_types=[
        pltpu.VMEM((_TOK,), jnp.int32),
        pltpu.VMEM((_TOK, D_EMBED), jnp.float32),
        pltpu.SemaphoreType.DMA,
    ],
)(_emb_body)


@jax.jit
def kernel(inp, emb_weight):
    idx = inp.reshape(-1).astype(jnp.int32)
    return _emb_call(idx, emb_weight)
